# Initial kernel scaffold; baseline (speedup 1.0000x reference)
import jax
import jax.numpy as jnp
from jax.experimental import pallas as pl


def kernel(x, conn, tables):
    raise NotImplementedError("write your pallas kernel here")


# speedup vs baseline: 7.1200x; 7.1200x over previous
import functools

import jax
import jax.numpy as jnp
from jax import lax
from jax.experimental import pallas as pl
from jax.experimental.pallas import tpu as pltpu
from jax.experimental.pallas import tpu_sc as plsc

_INPUT_BITS = 4096
_NUM_SUB = 4
_BITS_PER_SUB = 12
_TABLE = 1 << _BITS_PER_SUB
_NW = 32
_JPW = _INPUT_BITS // _NW
_L = 16
_NCH = _JPW // _L
_CONN_W = _INPUT_BITS * _BITS_PER_SUB
_CONN_PW = _JPW * _BITS_PER_SUB


def _sc_body(x_hbm, conn_hbm, tab_hbm, out_hbm,
             x_v, conn_v, idx_v, got_v, out_v, sem):
    wid = lax.axis_index("s") * 2 + lax.axis_index("c")
    base = wid * _JPW

    pltpu.sync_copy(x_hbm, x_v)
    for i in range(_NUM_SUB):
        pltpu.sync_copy(
            conn_hbm.at[pl.ds(i * _CONN_W + base * _BITS_PER_SUB, _CONN_PW)],
            conn_v.at[pl.ds(i * _CONN_PW, _CONN_PW)])

    lane = lax.iota(jnp.int32, _L)
    lane12 = lane * _BITS_PER_SUB

    for i in range(_NUM_SUB):
        for jc in range(_NCH):
            cbase = i * _CONN_PW + jc * _L * _BITS_PER_SUB + lane12
            addr = jnp.zeros((_L,), jnp.int32)
            for b in range(_BITS_PER_SUB):
                bits = plsc.load_gather(conn_v, [cbase + b])
                bit = plsc.load_gather(x_v, [bits])
                addr = addr + (bit << b)
            row0 = i * _INPUT_BITS + base + jc * _L
            flat = row0 * _TABLE + lane * _TABLE + addr
            idx_v[i, pl.ds(jc * _L, _L)] = flat

    copies = [pltpu.async_copy(tab_hbm.at[idx_v.at[i]], got_v.at[i], sem)
              for i in range(_NUM_SUB)]
    for c in copies:
        c.wait()

    for jc in range(_NCH):
        ones = jnp.zeros((_L,), jnp.int32)
        for i in range(_NUM_SUB):
            v = got_v[i, pl.ds(jc * _L, _L)]
            ones = ones + jnp.where(v > 0.5, 1, 0).astype(jnp.int32)
        out_v[pl.ds(jc * _L, _L)] = jnp.where(ones > 2, 1, 0).astype(jnp.int32)

    pltpu.sync_copy(out_v, out_hbm.at[pl.ds(base, _JPW)])


@functools.partial(
    pl.kernel,
    out_type=jax.ShapeDtypeStruct((_INPUT_BITS,), jnp.int32),
    mesh=plsc.VectorSubcoreMesh(core_axis_name="c", subcore_axis_name="s"),
    compiler_params=pltpu.CompilerParams(needs_layout_passes=False),
    scratch_types=[
        pltpu.VMEM((_INPUT_BITS,), jnp.int32),
        pltpu.VMEM((_NUM_SUB * _CONN_PW,), jnp.int32),
        pltpu.VMEM((_NUM_SUB, _JPW), jnp.int32),
        pltpu.VMEM((_NUM_SUB, _JPW), jnp.float32),
        pltpu.VMEM((_JPW,), jnp.int32),
        pltpu.SemaphoreType.DMA,
    ],
)
def _sc_kernel(x_hbm, conn_hbm, tab_hbm, out_hbm,
               x_v, conn_v, idx_v, got_v, out_v, sem):
    _sc_body(x_hbm, conn_hbm, tab_hbm, out_hbm,
             x_v, conn_v, idx_v, got_v, out_v, sem)


def kernel(x, conn, tables):
    out = _sc_kernel(x, conn.reshape(-1), tables.reshape(-1))
    return out.astype(jnp.uint8)
